# baseline (speedup 1.0000x reference)
import jax
import jax.numpy as jnp
from jax.experimental import pallas as pl
from jax.experimental.pallas import tpu as pltpu

_IN = 2048
_OUT = 2048
_TOPK = max(1, int(_OUT * 0.1))
_BT = 512
_NCHUNK = 16
_CW = _OUT // _NCHUNK

_DIMS = (((1,), (1,)), ((), ()))


def _float_keys(r):
    bits = jax.lax.bitcast_convert_type(r, jnp.int32)
    return bits ^ ((bits >> 31) & jnp.int32(0x7FFFFFFF))


def _search_step(keys, lo, hi):
    mid = (lo >> 1) + (hi >> 1) + ((lo | hi) & 1)
    cnt = jnp.sum((keys >= mid).astype(jnp.int32), axis=1, keepdims=True)
    ge = cnt >= _TOPK
    return jnp.where(ge, mid, lo), jnp.where(ge, hi, mid - 1)


def _frozen_chunk(j, xb, w_ref, b_ref, out_ref):
    wc = w_ref[j * _CW:(j + 1) * _CW, :]
    oc = jax.lax.dot_general(xb, wc, _DIMS,
                             preferred_element_type=jnp.float32)
    out_ref[:, j * _CW:(j + 1) * _CW] = oc + b_ref[:, j * _CW:(j + 1) * _CW]


def _fused_kernel(x_ref, wr_ref, br_ref, w_ref, b_ref, out_ref):
    xb = x_ref[...]
    r = jax.lax.dot_general(xb, wr_ref[...], _DIMS,
                            preferred_element_type=jnp.float32) + br_ref[...]
    keys = _float_keys(r)
    rows = keys.shape[0]

    keys16 = (keys >> 16).astype(jnp.int16)
    lo = jnp.full((rows, 1), -32768, jnp.int32)
    hi = jnp.full((rows, 1), 32767, jnp.int32)
    for j in range(_NCHUNK // 2):
        for _ in range(2):
            mid = (lo + hi + 1) >> 1
            cmp = keys16 >= mid.astype(jnp.int16)
            cnt = jnp.sum(cmp.astype(jnp.int32), axis=1, keepdims=True)
            ge = cnt >= _TOPK
            lo = jnp.where(ge, mid, lo)
            hi = jnp.where(ge, hi, mid - 1)
        _frozen_chunk(j, xb, w_ref, b_ref, out_ref)

    lo = lo << 16
    hi = lo + 0xFFFF
    for j in range(_NCHUNK // 2, _NCHUNK):
        lo, hi = _search_step(keys, lo, hi)
        lo, hi = _search_step(keys, lo, hi)
        _frozen_chunk(j, xb, w_ref, b_ref, out_ref)

    mask = (keys >= lo).astype(jnp.float32)
    out_ref[...] = out_ref[...] * mask


@jax.jit
def kernel(x, W, b, W_r, b_r):
    B, S, F = x.shape
    T = B * S
    xt = x.reshape(T, F)
    grid = (T // _BT,)
    out = pl.pallas_call(
        _fused_kernel,
        grid=grid,
        in_specs=[
            pl.BlockSpec((_BT, F), lambda i: (i, 0)),
            pl.BlockSpec((_OUT, F), lambda i: (0, 0)),
            pl.BlockSpec((1, _OUT), lambda i: (0, 0)),
            pl.BlockSpec((_OUT, F), lambda i: (0, 0)),
            pl.BlockSpec((1, _OUT), lambda i: (0, 0)),
        ],
        out_specs=pl.BlockSpec((_BT, _OUT), lambda i: (i, 0)),
        out_shape=jax.ShapeDtypeStruct((T, _OUT), jnp.float32),
        compiler_params=pltpu.CompilerParams(
            vmem_limit_bytes=100 * 1024 * 1024),
    )(xt, W_r, b_r.reshape(1, _OUT), W, b.reshape(1, _OUT))
    return out.reshape(B, S, _OUT)


# speedup vs baseline: 1.2285x; 1.2285x over previous
import jax
import jax.numpy as jnp
from jax.experimental import pallas as pl
from jax.experimental.pallas import tpu as pltpu

_IN = 2048
_OUT = 2048
_TOPK = max(1, int(_OUT * 0.1))
_BT = 512
_NCHUNK = 16
_CW = _OUT // _NCHUNK

_DIMS = (((1,), (1,)), ((), ()))


def _float_keys(r):
    bits = jax.lax.bitcast_convert_type(r, jnp.int32)
    return bits ^ ((bits >> 31) & jnp.int32(0x7FFFFFFF))


def _search_step(keys, lo, hi):
    mid = (lo >> 1) + (hi >> 1) + ((lo | hi) & 1)
    cnt = jnp.sum((keys >= mid).astype(jnp.int32), axis=1, keepdims=True)
    ge = cnt >= _TOPK
    return jnp.where(ge, mid, lo), jnp.where(ge, hi, mid - 1)


def _fused_kernel(x_ref, wr_ref, br_ref, w_ref, b_ref, out_ref):
    xb = x_ref[...]
    r = jax.lax.dot_general(xb, wr_ref[...], _DIMS,
                            preferred_element_type=jnp.float32) + br_ref[...]
    keys = _float_keys(r)
    rows = keys.shape[0]

    lo = jnp.full((rows, 1), jnp.iinfo(jnp.int32).min, jnp.int32)
    hi = jnp.full((rows, 1), jnp.iinfo(jnp.int32).max, jnp.int32)

    half = _NCHUNK // 2
    for j in range(half):
        for _ in range(4):
            lo, hi = _search_step(keys, lo, hi)
        wc = w_ref[j * _CW:(j + 1) * _CW, :]
        oc = jax.lax.dot_general(xb, wc, _DIMS,
                                 preferred_element_type=jnp.float32)
        out_ref[:, j * _CW:(j + 1) * _CW] = oc + b_ref[:, j * _CW:(j + 1) * _CW]

    for j in range(half, _NCHUNK):
        cs = slice(j * _CW, (j + 1) * _CW)
        wc = w_ref[cs, :]
        oc = jax.lax.dot_general(xb, wc, _DIMS,
                                 preferred_element_type=jnp.float32)
        maskc = (keys[:, cs] >= lo).astype(jnp.float32)
        out_ref[:, cs] = (oc + b_ref[:, cs]) * maskc

    cs = slice(0, half * _CW)
    mask = (keys[:, cs] >= lo).astype(jnp.float32)
    out_ref[:, cs] = out_ref[:, cs] * mask


@jax.jit
def kernel(x, W, b, W_r, b_r):
    B, S, F = x.shape
    T = B * S
    xt = x.reshape(T, F)
    grid = (T // _BT,)
    out = pl.pallas_call(
        _fused_kernel,
        grid=grid,
        in_specs=[
            pl.BlockSpec((_BT, F), lambda i: (i, 0)),
            pl.BlockSpec((_OUT, F), lambda i: (0, 0)),
            pl.BlockSpec((1, _OUT), lambda i: (0, 0)),
            pl.BlockSpec((_OUT, F), lambda i: (0, 0)),
            pl.BlockSpec((1, _OUT), lambda i: (0, 0)),
        ],
        out_specs=pl.BlockSpec((_BT, _OUT), lambda i: (i, 0)),
        out_shape=jax.ShapeDtypeStruct((T, _OUT), jnp.float32),
        compiler_params=pltpu.CompilerParams(
            vmem_limit_bytes=100 * 1024 * 1024),
    )(xt, W_r, b_r.reshape(1, _OUT), W, b.reshape(1, _OUT))
    return out.reshape(B, S, _OUT)
